# baseline (speedup 1.0000x reference)
import functools

import jax
import jax.numpy as jnp
from jax import lax
from jax.experimental import pallas as pl
from jax.experimental.pallas import tpu as pltpu
from jax.experimental.pallas import tpu_sc as plsc

NUM_FACTORS = 32
BATCH = 16384
NC, NS, L = 2, 16, 16
NW = NC * NS
BPW = BATCH // NW
CHUNK = 128
NCHUNK = BPW // CHUNK
NGROUP = BPW // L


def _rsqrt(x):
    i = lax.bitcast_convert_type(x, jnp.int32)
    y = lax.bitcast_convert_type(
        jnp.int32(0x5F3759DF) - lax.shift_right_logical(i, 1), jnp.float32)
    for _ in range(3):
        y = y * (1.5 - 0.5 * x * y * y)
    return y


def _sc_kernel(users_hbm, movies_hbm, utab_hbm, mtab_hbm, out_hbm,
               uidx, midx, urows, mrows, outv, sem):
    wid = lax.axis_index("s") * NC + lax.axis_index("c")
    base = wid * BPW

    for k in range(NCHUNK):
        pltpu.sync_copy(users_hbm.at[pl.ds(base + k * CHUNK, CHUNK)], uidx.at[k])
        pltpu.sync_copy(movies_hbm.at[pl.ds(base + k * CHUNK, CHUNK)], midx.at[k])

    handles = []
    for k in range(NCHUNK):
        handles.append(pltpu.async_copy(
            utab_hbm.at[uidx.at[k]], urows.at[pl.ds(k * CHUNK, CHUNK)], sem))
        handles.append(pltpu.async_copy(
            mtab_hbm.at[midx.at[k]], mrows.at[pl.ds(k * CHUNK, CHUNK)], sem))
    for h in handles:
        h.wait()

    iota = lax.iota(jnp.int32, L)
    zeros = jnp.zeros((L,), jnp.float32)

    def group_body(g, carry):
        rows = g * L + iota
        d, a2, b2 = zeros, zeros, zeros
        for j in range(NUM_FACTORS):
            cols = jnp.broadcast_to(jnp.int32(j), (L,))
            u = plsc.load_gather(urows, [rows, cols])
            m = plsc.load_gather(mrows, [rows, cols])
            d = d + u * m
            a2 = a2 + u * u
            b2 = b2 + m * m
        r = _rsqrt(a2 * b2)
        p2 = jnp.minimum(a2, 1.0) * jnp.minimum(b2, 1.0)
        pt = p2 * _rsqrt(p2)
        ratio = jnp.where(p2 < 1e-16, pt * 1e8, 1.0)
        outv[pl.ds(g * L, L)] = d * r * ratio * 2.0 + 3.0
        return carry

    lax.fori_loop(0, NGROUP, group_body, 0)
    pltpu.sync_copy(outv, out_hbm.at[pl.ds(base, BPW)])


@functools.partial(
    pl.kernel,
    mesh=plsc.VectorSubcoreMesh(core_axis_name="c", subcore_axis_name="s"),
    out_type=jax.ShapeDtypeStruct((BATCH,), jnp.float32),
    compiler_params=pltpu.CompilerParams(
        needs_layout_passes=False, use_tc_tiling_on_sc=False),
    scratch_types=[
        pltpu.VMEM((NCHUNK, CHUNK), jnp.int32),
        pltpu.VMEM((NCHUNK, CHUNK), jnp.int32),
        pltpu.VMEM((BPW, NUM_FACTORS), jnp.float32),
        pltpu.VMEM((BPW, NUM_FACTORS), jnp.float32),
        pltpu.VMEM((BPW,), jnp.float32),
        pltpu.SemaphoreType.DMA,
    ],
)
def _cosine_lookup(users, movies, utab, mtab, out, *scratch):
    _sc_kernel(users, movies, utab, mtab, out, *scratch)


def kernel(users, movies, user_table, movie_table):
    return _cosine_lookup(users.astype(jnp.int32), movies.astype(jnp.int32),
                          user_table, movie_table)
